# baseline (speedup 1.0000x reference)
import functools

import jax
import jax.numpy as jnp
from jax import lax
from jax.experimental import pallas as pl
from jax.experimental.pallas import tpu as pltpu
from jax.experimental.pallas import tpu_sc as plsc

HIDDEN = 128
NC, NS = 2, 16
NW = NC * NS
GCHUNK = 128
GPS = 2
SCHUNK = GCHUNK * GPS
NBUF = 3
LOOK = 1


def _make_lookup(B):
    b_per_w = B // NW
    n_chunks = b_per_w // SCHUNK
    n_grows = b_per_w // GCHUNK
    mesh = plsc.VectorSubcoreMesh(
        core_axis_name="c", subcore_axis_name="s", num_cores=NC, num_subcores=NS
    )

    @functools.partial(
        pl.kernel,
        out_type=jax.ShapeDtypeStruct((B, HIDDEN), jnp.float32),
        mesh=mesh,
        scratch_types=[
            pltpu.VMEM((n_grows, GCHUNK), jnp.int32),
            pltpu.VMEM((NBUF, SCHUNK, HIDDEN), jnp.float32),
            pltpu.SemaphoreType.DMA((NBUF,)),
            pltpu.SemaphoreType.DMA((NBUF,)),
        ],
    )
    def lookup(idx_hbm, table_hbm, out_hbm, idx_v, rows_v, gsem, ssem):
        wid = lax.axis_index("s") * NC + lax.axis_index("c")
        pltpu.sync_copy(idx_hbm.at[pl.ds(wid * n_grows, n_grows)], idx_v)
        base = wid * b_per_w

        def start_gathers(c, slot):
            for g in range(GPS):
                pltpu.async_copy(
                    table_hbm.at[idx_v.at[c * GPS + g]],
                    rows_v.at[slot, pl.ds(g * GCHUNK, GCHUNK)],
                    gsem.at[slot],
                )

        for j in range(LOOK):
            start_gathers(j, j)

        def body(i, carry):
            slot = lax.rem(i, NBUF)
            j = i + LOOK

            @pl.when(j < n_chunks)
            def _():
                jslot = lax.rem(j, NBUF)

                @pl.when(i >= NBUF - LOOK)
                def _():
                    pltpu.make_async_copy(
                        rows_v.at[jslot], out_hbm.at[pl.ds(0, SCHUNK)], ssem.at[jslot]
                    ).wait()

                start_gathers(j, jslot)

            pltpu.make_async_copy(
                out_hbm.at[pl.ds(0, SCHUNK)], rows_v.at[slot], gsem.at[slot]
            ).wait()
            pltpu.async_copy(
                rows_v.at[slot],
                out_hbm.at[pl.ds(base + i * SCHUNK, SCHUNK)],
                ssem.at[slot],
            )
            return carry

        lax.fori_loop(0, n_chunks, body, 0)

        for b in range(min(NBUF, n_chunks)):
            pltpu.make_async_copy(
                rows_v.at[b], out_hbm.at[pl.ds(0, SCHUNK)], ssem.at[b]
            ).wait()

    return lookup


def kernel(token_ids, table):
    B_, S_ = token_ids.shape
    flat = jnp.reshape(token_ids, (-1, GCHUNK)).astype(jnp.int32)
    out = _make_lookup(B_ * S_)(flat, table)
    return jnp.reshape(out, (B_, S_, HIDDEN))


# speedup vs baseline: 1.0019x; 1.0019x over previous
import functools

import jax
import jax.numpy as jnp
from jax import lax
from jax.experimental import pallas as pl
from jax.experimental.pallas import tpu as pltpu
from jax.experimental.pallas import tpu_sc as plsc

HIDDEN = 128
NC, NS = 2, 16
NW = NC * NS
CHUNK = 128
NBUF = 6
LOOK = 3


def _make_lookup(B):
    b_per_w = B // NW
    n_chunks = b_per_w // CHUNK
    mesh = plsc.VectorSubcoreMesh(
        core_axis_name="c", subcore_axis_name="s", num_cores=NC, num_subcores=NS
    )

    @functools.partial(
        pl.kernel,
        out_type=jax.ShapeDtypeStruct((B, HIDDEN), jnp.float32),
        mesh=mesh,
        scratch_types=[
            pltpu.VMEM((n_chunks, CHUNK), jnp.int32),
            pltpu.VMEM((NBUF, CHUNK, HIDDEN), jnp.float32),
            pltpu.SemaphoreType.DMA((NBUF,)),
            pltpu.SemaphoreType.DMA((NBUF,)),
        ],
    )
    def lookup(idx_hbm, table_hbm, out_hbm, idx_v, rows_v, gsem, ssem):
        wid = lax.axis_index("s") * NC + lax.axis_index("c")
        base = wid * n_chunks
        pltpu.sync_copy(idx_hbm.at[pl.ds(base, n_chunks)], idx_v)

        for j in range(LOOK):
            pltpu.async_copy(table_hbm.at[idx_v.at[j]], rows_v.at[j], gsem.at[j])

        def body(i, carry):
            slot = lax.rem(i, NBUF)
            j = i + LOOK

            @pl.when(j < n_chunks)
            def _():
                jslot = lax.rem(j, NBUF)

                @pl.when(i >= NBUF - LOOK)
                def _():
                    pltpu.make_async_copy(
                        rows_v.at[jslot], out_hbm.at[pl.ds(0, CHUNK)], ssem.at[jslot]
                    ).wait()

                pltpu.async_copy(
                    table_hbm.at[idx_v.at[j]], rows_v.at[jslot], gsem.at[jslot]
                )

            pltpu.make_async_copy(
                table_hbm.at[idx_v.at[slot]], rows_v.at[slot], gsem.at[slot]
            ).wait()
            pltpu.async_copy(
                rows_v.at[slot],
                out_hbm.at[pl.ds((base + i) * CHUNK, CHUNK)],
                ssem.at[slot],
            )
            return carry

        lax.fori_loop(0, n_chunks, body, 0)

        for b in range(NBUF):
            pltpu.make_async_copy(
                rows_v.at[b], out_hbm.at[pl.ds(0, CHUNK)], ssem.at[b]
            ).wait()

    return lookup


def kernel(token_ids, table):
    B_, S_ = token_ids.shape
    flat = jnp.reshape(token_ids, (-1, CHUNK)).astype(jnp.int32)
    out = _make_lookup(B_ * S_)(flat, table)
    return jnp.reshape(out, (B_, S_, HIDDEN))


# speedup vs baseline: 1.0041x; 1.0022x over previous
import functools

import jax
import jax.numpy as jnp
from jax import lax
from jax.experimental import pallas as pl
from jax.experimental.pallas import tpu as pltpu
from jax.experimental.pallas import tpu_sc as plsc

HIDDEN = 128
NC, NS = 2, 16
NW = NC * NS
CHUNK = 128
NBUF = 6
LOOK = 4


def _make_lookup(B):
    b_per_w = B // NW
    n_chunks = b_per_w // CHUNK
    mesh = plsc.VectorSubcoreMesh(
        core_axis_name="c", subcore_axis_name="s", num_cores=NC, num_subcores=NS
    )

    @functools.partial(
        pl.kernel,
        out_type=jax.ShapeDtypeStruct((B, HIDDEN), jnp.float32),
        mesh=mesh,
        scratch_types=[
            pltpu.VMEM((n_chunks, CHUNK), jnp.int32),
            pltpu.VMEM((NBUF, CHUNK, HIDDEN), jnp.float32),
            pltpu.SemaphoreType.DMA((NBUF,)),
            pltpu.SemaphoreType.DMA((NBUF,)),
        ],
    )
    def lookup(idx_hbm, table_hbm, out_hbm, idx_v, rows_v, gsem, ssem):
        wid = lax.axis_index("s") * NC + lax.axis_index("c")
        base = wid * n_chunks
        pltpu.sync_copy(idx_hbm.at[pl.ds(base, n_chunks)], idx_v)

        for j in range(LOOK):
            pltpu.async_copy(table_hbm.at[idx_v.at[j]], rows_v.at[j], gsem.at[j])

        def body(i, carry):
            slot = lax.rem(i, NBUF)
            j = i + LOOK

            @pl.when(j < n_chunks)
            def _():
                jslot = lax.rem(j, NBUF)

                @pl.when(i >= NBUF - LOOK)
                def _():
                    pltpu.make_async_copy(
                        rows_v.at[jslot], out_hbm.at[pl.ds(0, CHUNK)], ssem.at[jslot]
                    ).wait()

                pltpu.async_copy(
                    table_hbm.at[idx_v.at[j]], rows_v.at[jslot], gsem.at[jslot]
                )

            pltpu.make_async_copy(
                table_hbm.at[idx_v.at[slot]], rows_v.at[slot], gsem.at[slot]
            ).wait()
            pltpu.async_copy(
                rows_v.at[slot],
                out_hbm.at[pl.ds((base + i) * CHUNK, CHUNK)],
                ssem.at[slot],
            )
            return carry

        lax.fori_loop(0, n_chunks, body, 0)

        for b in range(NBUF):
            pltpu.make_async_copy(
                rows_v.at[b], out_hbm.at[pl.ds(0, CHUNK)], ssem.at[b]
            ).wait()

    return lookup


def kernel(token_ids, table):
    B_, S_ = token_ids.shape
    flat = jnp.reshape(token_ids, (-1, CHUNK)).astype(jnp.int32)
    out = _make_lookup(B_ * S_)(flat, table)
    return jnp.reshape(out, (B_, S_, HIDDEN))


# speedup vs baseline: 1.0475x; 1.0432x over previous
import functools

import jax
import jax.numpy as jnp
from jax import lax
from jax.experimental import pallas as pl
from jax.experimental.pallas import tpu as pltpu
from jax.experimental.pallas import tpu_sc as plsc

HIDDEN = 128
NC, NS = 2, 16
NW = NC * NS
CHUNK = 128
NBUF = 4
LOOK = 3
XBUF = 2


def _make_lookup(B):
    b_per_w = B // NW
    n_chunks = b_per_w // CHUNK
    mesh = plsc.VectorSubcoreMesh(
        core_axis_name="c", subcore_axis_name="s", num_cores=NC, num_subcores=NS
    )

    @functools.partial(
        pl.kernel,
        out_type=jax.ShapeDtypeStruct((B, HIDDEN), jnp.float32),
        mesh=mesh,
        scratch_types=[
            pltpu.VMEM((n_chunks, CHUNK), jnp.int32),
            pltpu.VMEM((NBUF, CHUNK, HIDDEN), jnp.float32),
            pltpu.VMEM_SHARED((NS, XBUF, CHUNK, HIDDEN), jnp.float32),
            pltpu.SemaphoreType.DMA((NBUF,)),
            pltpu.SemaphoreType.DMA((XBUF,)),
            pltpu.SemaphoreType.DMA((XBUF,)),
        ],
    )
    def lookup(idx_hbm, table_hbm, out_hbm, idx_v, rows_v, shr, gsem, xsem, ssem):
        wid = lax.axis_index("s") * NC + lax.axis_index("c")
        sid = lax.axis_index("s")
        pltpu.sync_copy(idx_hbm.at[pl.ds(wid * n_chunks, n_chunks)], idx_v)
        base = wid * b_per_w

        for j in range(LOOK):
            pltpu.async_copy(table_hbm.at[idx_v.at[j]], rows_v.at[j], gsem.at[j])

        def store_of(c):
            pslot = lax.rem(c, XBUF)
            pltpu.make_async_copy(
                rows_v.at[0], shr.at[sid, pslot], xsem.at[pslot]
            ).wait()
            pltpu.async_copy(
                shr.at[sid, pslot],
                out_hbm.at[pl.ds(base + c * CHUNK, CHUNK)],
                ssem.at[pslot],
            )

        def body(i, carry):
            slot = lax.rem(i, NBUF)
            xslot = lax.rem(i, XBUF)

            @pl.when(i >= 1)
            def _():
                store_of(i - 1)

            j = i + LOOK

            @pl.when(j < n_chunks)
            def _():
                pltpu.async_copy(
                    table_hbm.at[idx_v.at[j]],
                    rows_v.at[lax.rem(j, NBUF)],
                    gsem.at[lax.rem(j, NBUF)],
                )

            pltpu.make_async_copy(
                table_hbm.at[idx_v.at[slot]], rows_v.at[slot], gsem.at[slot]
            ).wait()

            @pl.when(i >= XBUF)
            def _():
                pltpu.make_async_copy(
                    shr.at[sid, xslot], out_hbm.at[pl.ds(0, CHUNK)], ssem.at[xslot]
                ).wait()

            pltpu.async_copy(rows_v.at[slot], shr.at[sid, xslot], xsem.at[xslot])
            return carry

        lax.fori_loop(0, n_chunks, body, 0)

        store_of(n_chunks - 1)
        for b in range(XBUF):
            pltpu.make_async_copy(
                shr.at[sid, b], out_hbm.at[pl.ds(0, CHUNK)], ssem.at[b]
            ).wait()

    return lookup


def kernel(token_ids, table):
    B_, S_ = token_ids.shape
    flat = jnp.reshape(token_ids, (-1, CHUNK)).astype(jnp.int32)
    out = _make_lookup(B_ * S_)(flat, table)
    return jnp.reshape(out, (B_, S_, HIDDEN))
